# baseline (speedup 1.0000x reference)
import dataclasses

import jax
import jax.numpy as jnp
from jax import lax
from jax.experimental import pallas as pl
from jax.experimental.pallas import tpu as pltpu
from jax.experimental.pallas import tpu_sc as plsc

B = 16384
F = 64
L = 16
NC, NS = 2, 16
NW = NC * NS
BPW = B // NW


def _cf_body(users_hbm, items_hbm, ue_hbm, ub_hbm, ie_hbm, ib_hbm, out_hbm,
             uidx_v, iidx_v, ue_v, ie_v, ub_v, ib_v, out_v,
             sem_u, sem_i, sem_ub, sem_ib):
    wid = lax.axis_index("s") * NC + lax.axis_index("c")
    base = wid * BPW

    pltpu.sync_copy(users_hbm.at[pl.ds(base, BPW)], uidx_v)
    pltpu.sync_copy(items_hbm.at[pl.ds(base, BPW)], iidx_v)

    cu = pltpu.async_copy(ue_hbm.at[uidx_v], ue_v, sem_u)
    ci = pltpu.async_copy(ie_hbm.at[iidx_v], ie_v, sem_i)
    cub = pltpu.async_copy(ub_hbm.at[uidx_v], ub_v, sem_ub)
    cib = pltpu.async_copy(ib_hbm.at[iidx_v], ib_v, sem_ib)
    cu.wait()
    ci.wait()
    cub.wait()
    cib.wait()

    lane = lax.broadcasted_iota(jnp.int32, (L,), 0)

    @pl.loop(0, BPW, step=L)
    def _(g):
        def row(j, res):
            b = g + j
            acc = ue_v[b, pl.ds(0, L)] * ie_v[b, pl.ds(0, L)]
            for c in range(1, F // L):
                acc = acc + ue_v[b, pl.ds(c * L, L)] * ie_v[b, pl.ds(c * L, L)]
            return jnp.where(lane == j, jnp.sum(acc), res)

        res = lax.fori_loop(0, L, row, jnp.zeros((L,), jnp.float32))
        out_v[pl.ds(g, L)] = res + ub_v[pl.ds(g, L)] + ib_v[pl.ds(g, L)]

    pltpu.sync_copy(out_v, out_hbm.at[pl.ds(base, BPW)])


def kernel(users, items, user_emb, user_bias, item_emb, item_bias):
    mesh = plsc.VectorSubcoreMesh(core_axis_name="c", subcore_axis_name="s")
    cp = pltpu.CompilerParams()
    fields = pltpu.CompilerParams.__dataclass_fields__
    if "needs_layout_passes" in fields:
        cp = dataclasses.replace(cp, needs_layout_passes=False)
    if "use_tc_tiling_on_sc" in fields:
        cp = dataclasses.replace(cp, use_tc_tiling_on_sc=False)
    k = pl.kernel(
        _cf_body,
        out_type=jax.ShapeDtypeStruct((B,), jnp.float32),
        mesh=mesh,
        compiler_params=cp,
        scratch_types=[
            pltpu.VMEM((BPW,), jnp.int32),
            pltpu.VMEM((BPW,), jnp.int32),
            pltpu.VMEM((BPW, F), jnp.float32),
            pltpu.VMEM((BPW, F), jnp.float32),
            pltpu.VMEM((BPW,), jnp.float32),
            pltpu.VMEM((BPW,), jnp.float32),
            pltpu.VMEM((BPW,), jnp.float32),
            pltpu.SemaphoreType.DMA,
            pltpu.SemaphoreType.DMA,
            pltpu.SemaphoreType.DMA,
            pltpu.SemaphoreType.DMA,
        ],
    )
    return k(users.astype(jnp.int32), items.astype(jnp.int32),
             user_emb, user_bias.reshape(-1), item_emb, item_bias.reshape(-1))
